# baseline (speedup 1.0000x reference)
import functools

import jax
import jax.numpy as jnp
from jax import lax
from jax.experimental import pallas as pl
from jax.experimental.pallas import tpu as pltpu
from jax.experimental.pallas import tpu_sc as plsc

_N = 10000
_E = 160000
_D = 256
_NW = 32
_C = 80
_CPW = 67
_LAST = _N + _E - _C


@functools.partial(
    pl.kernel,
    mesh=plsc.VectorSubcoreMesh(core_axis_name="c", subcore_axis_name="s"),
    out_type=jax.ShapeDtypeStruct((_N + _E, _D), jnp.float32),
    scratch_types=[
        pltpu.VMEM((2 * _C,), jnp.int32),
        pltpu.VMEM((2 * _C,), jnp.int32),
        pltpu.VMEM((2 * _C, _D), jnp.float32),
        pltpu.VMEM((2 * _C, _D), jnp.float32),
        pltpu.VMEM((_C, _D), jnp.float32),
        pltpu.VMEM((_C, _D), jnp.float32),
        pltpu.SemaphoreType.DMA,
        pltpu.SemaphoreType.DMA,
        pltpu.SemaphoreType.DMA,
        pltpu.SemaphoreType.DMA,
        pltpu.SemaphoreType.DMA,
        pltpu.SemaphoreType.DMA,
    ],
)
def _unpool_kernel(table, raw, out, rawa, rawb, bufa, bufb, stga, stgb,
                   isema, isemb, gsema, gsemb, ssema, ssemb):
    w = lax.axis_index("s") * 2 + lax.axis_index("c")

    sets = ((rawa, bufa, stga, isema, gsema, ssema),
            (rawb, bufb, stgb, isemb, gsemb, ssemb))

    def chunk_base(g):
        cid = w + g * _NW
        return jnp.minimum(cid * _C, _LAST)

    def rawload(b, g):
        rawv, _, _, isem, _, _ = sets[b]
        eoff = jnp.maximum(chunk_base(g) - _N, 0)
        off = pl.multiple_of(2 * eoff, 8)
        return pltpu.make_async_copy(
            raw.at[pl.ds(off, 2 * _C)], rawv, isem)

    def gathers(b):
        rawv, buf, _, _, gsem, _ = sets[b]
        c0 = pltpu.make_async_copy(
            table.at[rawv.at[pl.ds(0, _C)]], buf.at[pl.ds(0, _C)], gsem)
        c1 = pltpu.make_async_copy(
            table.at[rawv.at[pl.ds(_C, _C)]], buf.at[pl.ds(_C, _C)], gsem)
        return c0, c1

    def lincopy(b, g):
        _, _, stg, _, gsem, _ = sets[b]
        return pltpu.make_async_copy(
            table.at[pl.ds(chunk_base(g), _C)], stg, gsem)

    def fetch_start(b, g):
        @pl.when(chunk_base(g) >= _N)
        def _():
            c0, c1 = gathers(b)
            c0.start()
            c1.start()

        @pl.when(chunk_base(g) < _N)
        def _():
            lincopy(b, g).start()

    def fetch_wait(b, g):
        @pl.when(chunk_base(g) >= _N)
        def _():
            c0, c1 = gathers(b)
            c0.wait()
            c1.wait()

        @pl.when(chunk_base(g) < _N)
        def _():
            lincopy(b, g).wait()

    def store(b, g):
        _, _, stg, _, _, ssem = sets[b]
        return pltpu.make_async_copy(
            stg, out.at[pl.ds(chunk_base(g), _C)], ssem)

    for b in range(2):
        rawload(b, b).start()
    for b in range(2):
        rawload(b, b).wait()
        fetch_start(b, b)

    def chunk_step(g, carry):
        for b in range(2):
            @pl.when(g % 2 == b)
            def _():
                _, buf, stg, _, _, _ = sets[b]
                fetch_wait(b, g)

                @pl.when(g + 2 < _CPW)
                def _():
                    rawload(b, g + 2).start()

                @pl.when(chunk_base(g) >= _N)
                def _():
                    def row(r, c2):
                        for j in range(_D // 16):
                            sl = pl.ds(j * 16, 16)
                            stg[r, sl] = (buf[2 * r, sl]
                                          + buf[2 * r + 1, sl]) * 0.5
                        return c2

                    lax.fori_loop(0, _C, row, 0)

                store(b, g).start()

                @pl.when(g + 2 < _CPW)
                def _():
                    store(b, g).wait()
                    rawload(b, g + 2).wait()
                    fetch_start(b, g + 2)
        return carry

    lax.fori_loop(0, _CPW, chunk_step, 0)

    store(0, 0).wait()
    store(1, 0).wait()


def kernel(inputs, unpool_idx):
    table = inputs.reshape(_N, _D)
    raw = unpool_idx.astype(jnp.int32).reshape(2 * _E)
    out = _unpool_kernel(table, raw)
    return out[None]


# speedup vs baseline: 1.8551x; 1.8551x over previous
import functools

import jax
import jax.numpy as jnp
from jax import lax
from jax.experimental import pallas as pl
from jax.experimental.pallas import tpu as pltpu
from jax.experimental.pallas import tpu_sc as plsc

_N = 10000
_E = 160000
_D = 256
_NW = 32
_C = 120
_CPW = 45
_LAST = _N + _E - _C


@functools.partial(
    pl.kernel,
    mesh=plsc.VectorSubcoreMesh(core_axis_name="c", subcore_axis_name="s"),
    out_type=jax.ShapeDtypeStruct((_N + _E, _D), jnp.float32),
    scratch_types=[
        pltpu.VMEM((2 * _C,), jnp.int32),
        pltpu.VMEM((2 * _C,), jnp.int32),
        pltpu.VMEM((_C, _D // 2), jnp.int32),
        pltpu.VMEM((_C, _D // 2), jnp.int32),
        pltpu.VMEM((_C, _D // 2), jnp.int32),
        pltpu.VMEM((_C, _D // 2), jnp.int32),
        pltpu.VMEM((_C, _D), jnp.float32),
        pltpu.VMEM((_C, _D), jnp.float32),
        pltpu.SemaphoreType.DMA,
        pltpu.SemaphoreType.DMA,
        pltpu.SemaphoreType.DMA,
        pltpu.SemaphoreType.DMA,
        pltpu.SemaphoreType.DMA,
        pltpu.SemaphoreType.DMA,
    ],
)
def _unpool_kernel(table, idx0, idx1, out, idxa, idxb,
                   rows0a, rows1a, rows0b, rows1b, stga, stgb,
                   isema, isemb, gsema, gsemb, ssema, ssemb):
    w = lax.axis_index("s") * 2 + lax.axis_index("c")

    sets = ((idxa, rows0a, rows1a, stga, isema, gsema, ssema),
            (idxb, rows0b, rows1b, stgb, isemb, gsemb, ssemb))

    def chunk_base(g):
        cid = w + g * _NW
        return jnp.minimum(cid * _C, _LAST)

    def idxload(b, g):
        idxv, _, _, _, isem, _, _ = sets[b]
        base = chunk_base(g)
        c0 = pltpu.make_async_copy(
            idx0.at[pl.ds(base, _C)], idxv.at[pl.ds(0, _C)], isem)
        c1 = pltpu.make_async_copy(
            idx1.at[pl.ds(base, _C)], idxv.at[pl.ds(_C, _C)], isem)
        return c0, c1

    def gathers(b):
        idxv, rows0, rows1, _, _, gsem, _ = sets[b]
        c0 = pltpu.make_async_copy(
            table.at[idxv.at[pl.ds(0, _C)]], rows0, gsem)
        c1 = pltpu.make_async_copy(
            table.at[idxv.at[pl.ds(_C, _C)]], rows1, gsem)
        return c0, c1

    def store(b, g):
        _, _, _, stg, _, _, ssem = sets[b]
        return pltpu.make_async_copy(
            stg, out.at[pl.ds(chunk_base(g), _C)], ssem)

    def widen(vi):
        lo = lax.bitcast_convert_type(vi << 16, jnp.float32)
        hi = lax.bitcast_convert_type(vi & jnp.int32(-65536), jnp.float32)
        return lo, hi

    for b in range(2):
        i0, i1 = idxload(b, b)
        i0.start()
        i1.start()
    for b in range(2):
        i0, i1 = idxload(b, b)
        i0.wait()
        i1.wait()
        c0, c1 = gathers(b)
        c0.start()
        c1.start()

    def chunk_step(g, carry):
        for b in range(2):
            @pl.when(g % 2 == b)
            def _():
                _, rows0, rows1, stg, _, _, _ = sets[b]
                c0, c1 = gathers(b)
                c0.wait()
                c1.wait()

                @pl.when(g + 2 < _CPW)
                def _():
                    i0, i1 = idxload(b, g + 2)
                    i0.start()
                    i1.start()

                @pl.when(g >= 2)
                def _():
                    store(b, g - 2).wait()

                def row(r, c2):
                    for j in range(_D // 32):
                        sl = pl.ds(16 * j, 16)
                        a_lo, a_hi = widen(rows0[r, sl])
                        b_lo, b_hi = widen(rows1[r, sl])
                        stg[r, pl.ds(32 * j, 16)] = (a_lo + b_lo) * 0.5
                        stg[r, pl.ds(32 * j + 16, 16)] = (a_hi + b_hi) * 0.5
                    return c2

                lax.fori_loop(0, _C, row, 0)
                store(b, g).start()

                @pl.when(g + 2 < _CPW)
                def _():
                    i0, i1 = idxload(b, g + 2)
                    i0.wait()
                    i1.wait()
                    n0, n1 = gathers(b)
                    n0.start()
                    n1.start()
        return carry

    lax.fori_loop(0, _CPW, chunk_step, 0)

    store(0, 0).wait()
    store(1, 0).wait()


def kernel(inputs, unpool_idx):
    table = inputs.reshape(_N, _D)
    tbf = jax.lax.bitcast_convert_type(
        table.astype(jnp.bfloat16).reshape(_N, _D // 32, 2, 16)
        .transpose(0, 1, 3, 2),
        jnp.int32).reshape(_N, _D // 2)
    idx = unpool_idx.astype(jnp.int32)
    self_ids = jnp.arange(_N, dtype=jnp.int32)
    idx0 = jnp.concatenate([self_ids, idx[:, 0]])
    idx1 = jnp.concatenate([self_ids, idx[:, 1]])
    out = _unpool_kernel(tbf, idx0, idx1)
    return out[None]


# speedup vs baseline: 1.8758x; 1.0112x over previous
import functools

import jax
import jax.numpy as jnp
from jax import lax
from jax.experimental import pallas as pl
from jax.experimental.pallas import tpu as pltpu
from jax.experimental.pallas import tpu_sc as plsc

_N = 10000
_E = 160000
_D = 256
_NW = 32
_C = 120
_CPW = 45
_LAST = _N + _E - _C


@functools.partial(
    pl.kernel,
    mesh=plsc.VectorSubcoreMesh(core_axis_name="c", subcore_axis_name="s"),
    out_type=jax.ShapeDtypeStruct((_N + _E, _D), jnp.float32),
    scratch_types=[
        pltpu.VMEM((2 * _C,), jnp.int32),
        pltpu.VMEM((2 * _C,), jnp.int32),
        pltpu.VMEM((_C, _D // 2), jnp.int32),
        pltpu.VMEM((_C, _D // 2), jnp.int32),
        pltpu.VMEM((_C, _D // 2), jnp.int32),
        pltpu.VMEM((_C, _D // 2), jnp.int32),
        pltpu.VMEM((_C, _D), jnp.float32),
        pltpu.VMEM((_C, _D), jnp.float32),
        pltpu.SemaphoreType.DMA,
        pltpu.SemaphoreType.DMA,
        pltpu.SemaphoreType.DMA,
        pltpu.SemaphoreType.DMA,
        pltpu.SemaphoreType.DMA,
        pltpu.SemaphoreType.DMA,
    ],
)
def _unpool_kernel(table, idx0, idx1, out, idxa, idxb,
                   rows0a, rows1a, rows0b, rows1b, stga, stgb,
                   isema, isemb, gsema, gsemb, ssema, ssemb):
    w = lax.axis_index("s") * 2 + lax.axis_index("c")

    sets = ((idxa, rows0a, rows1a, stga, isema, gsema, ssema),
            (idxb, rows0b, rows1b, stgb, isemb, gsemb, ssemb))

    def chunk_base(g):
        cid = w + g * _NW
        return jnp.minimum(cid * _C, _LAST)

    def idxload(b, g):
        idxv, _, _, _, isem, _, _ = sets[b]
        base = chunk_base(g)
        c0 = pltpu.make_async_copy(
            idx0.at[pl.ds(base, _C)], idxv.at[pl.ds(0, _C)], isem)
        c1 = pltpu.make_async_copy(
            idx1.at[pl.ds(base, _C)], idxv.at[pl.ds(_C, _C)], isem)
        return c0, c1

    def gathers(b):
        idxv, rows0, rows1, _, _, gsem, _ = sets[b]
        c0 = pltpu.make_async_copy(
            table.at[idxv.at[pl.ds(0, _C)]], rows0, gsem)
        c1 = pltpu.make_async_copy(
            table.at[idxv.at[pl.ds(_C, _C)]], rows1, gsem)
        return c0, c1

    def store(b, g):
        _, _, _, stg, _, _, ssem = sets[b]
        return pltpu.make_async_copy(
            stg, out.at[pl.ds(chunk_base(g), _C)], ssem)

    def widen(vi):
        lo = lax.bitcast_convert_type(vi << 16, jnp.float32)
        hi = lax.bitcast_convert_type(vi & jnp.int32(-65536), jnp.float32)
        return lo, hi

    for b in range(2):
        i0, i1 = idxload(b, b)
        i0.start()
        i1.start()
    for b in range(2):
        i0, i1 = idxload(b, b)
        i0.wait()
        i1.wait()
        c0, c1 = gathers(b)
        c0.start()
        c1.start()

    def chunk_step(g, carry):
        for b in range(2):
            @pl.when(g % 2 == b)
            def _():
                _, rows0, rows1, stg, _, _, _ = sets[b]
                c0, c1 = gathers(b)
                c0.wait()
                c1.wait()

                @pl.when(g + 2 < _CPW)
                def _():
                    i0, i1 = idxload(b, g + 2)
                    i0.start()
                    i1.start()

                @pl.when(g >= 2)
                def _():
                    store(b, g - 2).wait()

                def row(r, c2):
                    for j in range(_D // 32):
                        sl = pl.ds(16 * j, 16)
                        a_lo, a_hi = widen(rows0[r, sl])
                        b_lo, b_hi = widen(rows1[r, sl])
                        stg[r, pl.ds(32 * j, 16)] = (a_lo + b_lo) * 0.5
                        stg[r, pl.ds(32 * j + 16, 16)] = (a_hi + b_hi) * 0.5
                    return c2

                lax.fori_loop(0, _C, row, 0)
                store(b, g).start()

                @pl.when(g + 2 < _CPW)
                def _():
                    i0, i1 = idxload(b, g + 2)
                    i0.wait()
                    i1.wait()
                    n0, n1 = gathers(b)
                    n0.start()
                    n1.start()
        return carry

    lax.fori_loop(0, _CPW, chunk_step, 0)

    store(0, 0).wait()
    store(1, 0).wait()


def kernel(inputs, unpool_idx):
    table = inputs.reshape(_N, _D)
    tbits = jax.lax.bitcast_convert_type(
        table.astype(jnp.bfloat16), jnp.uint16).reshape(_N, _D // 32, 2, 16)
    lo = tbits[:, :, 0, :].astype(jnp.uint32)
    hi = tbits[:, :, 1, :].astype(jnp.uint32)
    tbf = jax.lax.bitcast_convert_type(
        (hi << 16) | lo, jnp.int32).reshape(_N, _D // 2)
    idx = unpool_idx.astype(jnp.int32)
    self_ids = jnp.arange(_N, dtype=jnp.int32)
    idx0 = jnp.concatenate([self_ids, idx[:, 0]])
    idx1 = jnp.concatenate([self_ids, idx[:, 1]])
    out = _unpool_kernel(tbf, idx0, idx1)
    return out[None]


# speedup vs baseline: 3.1714x; 1.6907x over previous
import functools

import jax
import jax.numpy as jnp
from jax import lax
from jax.experimental import pallas as pl
from jax.experimental.pallas import tpu as pltpu
from jax.experimental.pallas import tpu_sc as plsc

_N = 10000
_E = 160000
_D = 256
_NW = 32
_C = 120
_S0 = 64
_S1 = _C - _S0
_CPW = 45
_LAST = _N + _E - _C


@functools.partial(
    pl.kernel,
    mesh=plsc.VectorSubcoreMesh(core_axis_name="c", subcore_axis_name="s"),
    out_type=jax.ShapeDtypeStruct((_N + _E, _D), jnp.float32),
    scratch_types=[
        pltpu.VMEM((2 * _C,), jnp.int32),
        pltpu.VMEM((2 * _C,), jnp.int32),
        pltpu.VMEM((_C, _D), jnp.float32),
        pltpu.VMEM((_C, _D), jnp.float32),
        pltpu.VMEM((_C, _D), jnp.float32),
        pltpu.VMEM((_C, _D), jnp.float32),
        pltpu.SemaphoreType.DMA,
        pltpu.SemaphoreType.DMA,
        pltpu.SemaphoreType.DMA,
        pltpu.SemaphoreType.DMA,
        pltpu.SemaphoreType.DMA,
        pltpu.SemaphoreType.DMA,
    ],
)
def _unpool_kernel(table, idx0, idx1, out, idxa, idxb,
                   rows0a, rows1a, rows0b, rows1b,
                   isema, isemb, gsema, gsemb, ssema, ssemb):
    w = lax.axis_index("s") * 2 + lax.axis_index("c")

    sets = ((idxa, rows0a, rows1a, isema, gsema, ssema),
            (idxb, rows0b, rows1b, isemb, gsemb, ssemb))

    def chunk_base(g):
        cid = w + g * _NW
        return jnp.minimum(cid * _C, _LAST)

    def idxload(b, g):
        idxv, _, _, isem, _, _ = sets[b]
        base = chunk_base(g)
        c0 = pltpu.make_async_copy(
            idx0.at[pl.ds(base, _C)], idxv.at[pl.ds(0, _C)], isem)
        c1 = pltpu.make_async_copy(
            idx1.at[pl.ds(base, _C)], idxv.at[pl.ds(_C, _C)], isem)
        return c0, c1

    def gathers(b):
        idxv, rows0, rows1, _, gsem, _ = sets[b]
        return (
            pltpu.make_async_copy(
                table.at[idxv.at[pl.ds(0, _S0)]],
                rows0.at[pl.ds(0, _S0)], gsem),
            pltpu.make_async_copy(
                table.at[idxv.at[pl.ds(_S0, _S1)]],
                rows0.at[pl.ds(_S0, _S1)], gsem),
            pltpu.make_async_copy(
                table.at[idxv.at[pl.ds(_C, _S0)]],
                rows1.at[pl.ds(0, _S0)], gsem),
            pltpu.make_async_copy(
                table.at[idxv.at[pl.ds(_C + _S0, _S1)]],
                rows1.at[pl.ds(_S0, _S1)], gsem),
        )

    def store(b, g):
        _, rows0, _, _, _, ssem = sets[b]
        return pltpu.make_async_copy(
            rows0, out.at[pl.ds(chunk_base(g), _C)], ssem)

    for b in range(2):
        i0, i1 = idxload(b, b)
        i0.start()
        i1.start()
    for b in range(2):
        i0, i1 = idxload(b, b)
        i0.wait()
        i1.wait()
        for c in gathers(b):
            c.start()

    def chunk_step(g, carry):
        for b in range(2):
            @pl.when(g % 2 == b)
            def _():
                _, rows0, rows1, _, _, _ = sets[b]
                for c in gathers(b):
                    c.wait()

                @pl.when(g + 2 < _CPW)
                def _():
                    i0, i1 = idxload(b, g + 2)
                    i0.start()
                    i1.start()

                def row(r, c2):
                    for j in range(_D // 16):
                        sl = pl.ds(j * 16, 16)
                        rows0[r, sl] = (rows0[r, sl] + rows1[r, sl]) * 0.5
                    return c2

                lax.fori_loop(0, _C, row, 0)
                store(b, g).start()

                @pl.when(g + 2 < _CPW)
                def _():
                    store(b, g).wait()
                    i0, i1 = idxload(b, g + 2)
                    i0.wait()
                    i1.wait()
                    for c in gathers(b):
                        c.start()
        return carry

    lax.fori_loop(0, _CPW, chunk_step, 0)

    store(0, 0).wait()
    store(1, 0).wait()


def kernel(inputs, unpool_idx):
    table = inputs.reshape(_N, _D)
    idx = unpool_idx.astype(jnp.int32)
    self_ids = jnp.arange(_N, dtype=jnp.int32)
    idx0 = jnp.concatenate([self_ids, idx[:, 0]])
    idx1 = jnp.concatenate([self_ids, idx[:, 1]])
    out = _unpool_kernel(table, idx0, idx1)
    return out[None]


# speedup vs baseline: 3.2382x; 1.0211x over previous
import functools

import jax
import jax.numpy as jnp
from jax import lax
from jax.experimental import pallas as pl
from jax.experimental.pallas import tpu as pltpu
from jax.experimental.pallas import tpu_sc as plsc

_N = 10000
_E = 160000
_D = 256
_NW = 32
_C = 120
_S0 = 64
_S1 = _C - _S0
_CPW = 45
_LAST = _N + _E - _C


@functools.partial(
    pl.kernel,
    mesh=plsc.VectorSubcoreMesh(core_axis_name="c", subcore_axis_name="s"),
    out_type=jax.ShapeDtypeStruct((_N + _E, _D), jnp.float32),
    scratch_types=[
        pltpu.VMEM((2 * _C,), jnp.int32),
        pltpu.VMEM((2 * _C,), jnp.int32),
        pltpu.VMEM((_C, _D), jnp.float32),
        pltpu.VMEM((_C, _D), jnp.float32),
        pltpu.VMEM((_C, _D), jnp.float32),
        pltpu.VMEM((_C, _D), jnp.float32),
        pltpu.SemaphoreType.DMA,
        pltpu.SemaphoreType.DMA,
        pltpu.SemaphoreType.DMA,
        pltpu.SemaphoreType.DMA,
        pltpu.SemaphoreType.DMA,
        pltpu.SemaphoreType.DMA,
    ],
)
def _unpool_kernel(table, idx0, idx1, out, idxa, idxb,
                   rows0a, rows1a, rows0b, rows1b,
                   isema, isemb, gsema, gsemb, ssema, ssemb):
    w = lax.axis_index("s") * 2 + lax.axis_index("c")

    sets = ((idxa, rows0a, rows1a, isema, gsema, ssema),
            (idxb, rows0b, rows1b, isemb, gsemb, ssemb))

    def chunk_base(g):
        cid = w + g * _NW
        return jnp.minimum(cid * _C, _LAST)

    def idxload(b, g):
        idxv, _, _, isem, _, _ = sets[b]
        base = chunk_base(g)
        c0 = pltpu.make_async_copy(
            idx0.at[pl.ds(base, _C)], idxv.at[pl.ds(0, _C)], isem)
        c1 = pltpu.make_async_copy(
            idx1.at[pl.ds(base, _C)], idxv.at[pl.ds(_C, _C)], isem)
        return c0, c1

    def gathers(b):
        idxv, rows0, rows1, _, gsem, _ = sets[b]
        return (
            pltpu.make_async_copy(
                table.at[idxv.at[pl.ds(0, _S0)]],
                rows0.at[pl.ds(0, _S0)], gsem),
            pltpu.make_async_copy(
                table.at[idxv.at[pl.ds(_S0, _S1)]],
                rows0.at[pl.ds(_S0, _S1)], gsem),
            pltpu.make_async_copy(
                table.at[idxv.at[pl.ds(_C, _S0)]],
                rows1.at[pl.ds(0, _S0)], gsem),
            pltpu.make_async_copy(
                table.at[idxv.at[pl.ds(_C + _S0, _S1)]],
                rows1.at[pl.ds(_S0, _S1)], gsem),
        )

    def lincopy(b, g):
        _, rows0, _, _, gsem, _ = sets[b]
        return pltpu.make_async_copy(
            table.at[pl.ds(chunk_base(g), _C)], rows0, gsem)

    def is_orig(g):
        return chunk_base(g) + _C <= _N

    def fetch_start(b, g):
        @pl.when(is_orig(g))
        def _():
            lincopy(b, g).start()

        @pl.when(jnp.logical_not(is_orig(g)))
        def _():
            for c in gathers(b):
                c.start()

    def fetch_wait(b, g):
        @pl.when(is_orig(g))
        def _():
            lincopy(b, g).wait()

        @pl.when(jnp.logical_not(is_orig(g)))
        def _():
            for c in gathers(b):
                c.wait()

    def store(b, g):
        _, rows0, _, _, _, ssem = sets[b]
        return pltpu.make_async_copy(
            rows0, out.at[pl.ds(chunk_base(g), _C)], ssem)

    for b in range(2):
        i0, i1 = idxload(b, b)
        i0.start()
        i1.start()
    for b in range(2):
        i0, i1 = idxload(b, b)
        i0.wait()
        i1.wait()
        fetch_start(b, b)

    def chunk_step(g, carry):
        for b in range(2):
            @pl.when(g % 2 == b)
            def _():
                _, rows0, rows1, _, _, _ = sets[b]
                fetch_wait(b, g)

                @pl.when(g + 2 < _CPW)
                def _():
                    i0, i1 = idxload(b, g + 2)
                    i0.start()
                    i1.start()

                @pl.when(jnp.logical_not(is_orig(g)))
                def _():
                    def row(r, c2):
                        for j in range(_D // 16):
                            sl = pl.ds(j * 16, 16)
                            rows0[r, sl] = (rows0[r, sl]
                                            + rows1[r, sl]) * 0.5
                        return c2

                    lax.fori_loop(0, _C, row, 0)

                store(b, g).start()

                @pl.when(g + 2 < _CPW)
                def _():
                    store(b, g).wait()
                    i0, i1 = idxload(b, g + 2)
                    i0.wait()
                    i1.wait()
                    fetch_start(b, g + 2)
        return carry

    lax.fori_loop(0, _CPW, chunk_step, 0)

    store(0, 0).wait()
    store(1, 0).wait()


def kernel(inputs, unpool_idx):
    table = inputs.reshape(_N, _D)
    idx = unpool_idx.astype(jnp.int32)
    self_ids = jnp.arange(_N, dtype=jnp.int32)
    idx0 = jnp.concatenate([self_ids, idx[:, 0]])
    idx1 = jnp.concatenate([self_ids, idx[:, 1]])
    out = _unpool_kernel(table, idx0, idx1)
    return out[None]
